# baseline (speedup 1.0000x reference)
import functools

import jax
import jax.numpy as jnp
from jax.experimental import pallas as pl


def _pack_kernel(nf_ref, w_ref, m_ref, r_ref):
    w = w_ref[...]
    b_dim = nf_ref.shape[0]
    d_idx = jax.lax.broadcasted_iota(jnp.int32, (1, 6), 1)
    wm = 1 << d_idx
    wr = 1 << ((d_idx + 3) % 6)
    for b in range(b_dim):
        act = jnp.dot(nf_ref[b], w, preferred_element_type=jnp.float32)
        pht = (act > 0).astype(jnp.int32)
        m_ref[b, :] = jnp.sum(pht * wm, axis=1)
        r_ref[b, :] = jnp.sum(pht * wr, axis=1)


def _main_kernel(dir_ref, m_ref, r_ref, out_ref):
    d = dir_ref[...]
    b_dim = out_ref.shape[0]
    for b in range(b_dim):
        mb = m_ref[b, :][:, None]
        rb = r_ref[b, :][None, :]
        combined = mb & rb
        out_ref[b] = ((combined >> d) & 1).astype(jnp.float32)


@functools.partial(jax.jit, static_argnames=())
def kernel(node_features, direction_matrix, port_feature_mask):
    B, N, F = node_features.shape
    dir32 = direction_matrix.astype(jnp.int32)

    m, r = pl.pallas_call(
        _pack_kernel,
        out_shape=(
            jax.ShapeDtypeStruct((B, N), jnp.int32),
            jax.ShapeDtypeStruct((B, N), jnp.int32),
        ),
    )(node_features, port_feature_mask)

    BI, BJ = 128, 2048
    grid = (N // BI, N // BJ)
    out = pl.pallas_call(
        _main_kernel,
        grid=grid,
        in_specs=[
            pl.BlockSpec((BI, BJ), lambda i, j: (i, j)),
            pl.BlockSpec((B, BI), lambda i, j: (0, i)),
            pl.BlockSpec((B, BJ), lambda i, j: (0, j)),
        ],
        out_specs=pl.BlockSpec((B, BI, BJ), lambda i, j: (0, i, j)),
        out_shape=jax.ShapeDtypeStruct((B, N, N), jnp.float32),
    )(dir32, m, r)
    return out


# speedup vs baseline: 1.0978x; 1.0978x over previous
import functools

import jax
import jax.numpy as jnp
from jax.experimental import pallas as pl
from jax.experimental.pallas import tpu as pltpu


def _fused_kernel(nf_ref, w_ref, dir_ref, out_ref, m_ref, r_ref):
    i = pl.program_id(0)
    b_dim = out_ref.shape[0]
    bi = dir_ref.shape[0]

    @pl.when(i == 0)
    def _pack():
        w = w_ref[...]
        d_idx = jax.lax.broadcasted_iota(jnp.int32, (1, 6), 1)
        wm = 1 << d_idx
        wr = 1 << ((d_idx + 3) % 6)
        for b in range(b_dim):
            act = jnp.dot(nf_ref[b], w, preferred_element_type=jnp.float32)
            pht = (act > 0).astype(jnp.int32)
            m_ref[b, :] = jnp.sum(pht * wm, axis=1)
            r_ref[b, :] = jnp.sum(pht * wr, axis=1)

    d = dir_ref[...]
    for b in range(b_dim):
        mb = m_ref[b, pl.ds(i * bi, bi)][:, None]
        rb = r_ref[b, :][None, :]
        combined = mb & rb
        out_ref[b] = ((combined >> d) & 1).astype(jnp.float32)


@functools.partial(jax.jit, static_argnames=())
def kernel(node_features, direction_matrix, port_feature_mask):
    B, N, F = node_features.shape
    dir32 = direction_matrix.astype(jnp.int32)

    BI = 512
    grid = (N // BI,)
    out = pl.pallas_call(
        _fused_kernel,
        grid=grid,
        in_specs=[
            pl.BlockSpec((B, N, F), lambda i: (0, 0, 0)),
            pl.BlockSpec((F, 6), lambda i: (0, 0)),
            pl.BlockSpec((BI, N), lambda i: (i, 0)),
        ],
        out_specs=pl.BlockSpec((B, BI, N), lambda i: (0, i, 0)),
        out_shape=jax.ShapeDtypeStruct((B, N, N), jnp.float32),
        scratch_shapes=[
            pltpu.VMEM((B, N), jnp.int32),
            pltpu.VMEM((B, N), jnp.int32),
        ],
    )(node_features, port_feature_mask, dir32)
    return out


# speedup vs baseline: 1.2469x; 1.1358x over previous
import functools

import jax
import jax.numpy as jnp
from jax.experimental import pallas as pl
from jax.experimental.pallas import tpu as pltpu


def _fused_kernel(nf_ref, w_ref, dir_ref, out_ref, m_ref, r_ref):
    i = pl.program_id(0)
    b_dim = out_ref.shape[0]
    bi = dir_ref.shape[0]

    n = r_ref.shape[-1]

    @pl.when(i == 0)
    def _pack():
        w = w_ref[...]
        d_idx = jax.lax.broadcasted_iota(jnp.int32, (6, 2), 0)
        col = jax.lax.broadcasted_iota(jnp.int32, (6, 2), 1)
        shift = jnp.where(col == 0, d_idx, (d_idx + 3) % 6)
        wmr = (1 << shift).astype(jnp.float32)
        for b in range(b_dim):
            act = jnp.dot(nf_ref[b], w, preferred_element_type=jnp.float32)
            phtf = (act > 0).astype(jnp.float32)
            pk = jnp.dot(phtf, wmr, preferred_element_type=jnp.float32)
            pk = pk.astype(jnp.int32)
            m_ref[b] = pk[:, 0:1]
            r_ref[b] = pk[:, 1:2].reshape(1, n)

    d = dir_ref[...]
    for b in range(b_dim):
        mb = m_ref[b, pl.ds(i * bi, bi), :]
        rb = r_ref[b]
        combined = mb & rb
        out_ref[b] = ((combined >> d) & 1).astype(jnp.float32)


@functools.partial(jax.jit, static_argnames=())
def kernel(node_features, direction_matrix, port_feature_mask):
    B, N, F = node_features.shape
    dir32 = direction_matrix.astype(jnp.int32)

    BI = 512
    grid = (N // BI,)
    out = pl.pallas_call(
        _fused_kernel,
        grid=grid,
        in_specs=[
            pl.BlockSpec((B, N, F), lambda i: (0, 0, 0)),
            pl.BlockSpec((F, 6), lambda i: (0, 0)),
            pl.BlockSpec((BI, N), lambda i: (i, 0)),
        ],
        out_specs=pl.BlockSpec((B, BI, N), lambda i: (0, i, 0)),
        out_shape=jax.ShapeDtypeStruct((B, N, N), jnp.float32),
        scratch_shapes=[
            pltpu.VMEM((B, N, 1), jnp.int32),
            pltpu.VMEM((B, 1, N), jnp.int32),
        ],
    )(node_features, port_feature_mask, dir32)
    return out


# speedup vs baseline: 1.2984x; 1.0413x over previous
import functools

import jax
import jax.numpy as jnp
from jax.experimental import pallas as pl
from jax.experimental.pallas import tpu as pltpu


def _fused_kernel(nf_ref, w_ref, dir_ref, out_ref, m_ref, r_ref):
    i = pl.program_id(0)
    b_dim = out_ref.shape[0]
    bi = dir_ref.shape[0]

    n = r_ref.shape[-1]

    @pl.when(i == 0)
    def _pack():
        w = w_ref[...]
        d_idx = jax.lax.broadcasted_iota(jnp.int32, (6, 2), 0)
        col = jax.lax.broadcasted_iota(jnp.int32, (6, 2), 1)
        shift = jnp.where(col == 0, d_idx, (d_idx + 3) % 6)
        wmr = (1 << shift).astype(jnp.float32)
        for b in range(b_dim):
            act = jnp.dot(nf_ref[b], w, preferred_element_type=jnp.float32)
            phtf = (act > 0).astype(jnp.float32)
            pk = jnp.dot(phtf, wmr, preferred_element_type=jnp.float32)
            pk = pk.astype(jnp.int32)
            m_ref[b] = pk[:, 0:1]
            r_ref[b] = jnp.transpose(pk[:, 1:2])

    d = dir_ref[...]
    for b in range(b_dim):
        mb = m_ref[b, pl.ds(i * bi, bi), :]
        rb = r_ref[b]
        combined = mb & rb
        out_ref[b] = ((combined >> d) & 1).astype(jnp.float32)


@functools.partial(jax.jit, static_argnames=())
def kernel(node_features, direction_matrix, port_feature_mask):
    B, N, F = node_features.shape
    dir32 = direction_matrix.astype(jnp.int32)

    BI = 512
    grid = (N // BI,)
    out = pl.pallas_call(
        _fused_kernel,
        grid=grid,
        in_specs=[
            pl.BlockSpec((B, N, F), lambda i: (0, 0, 0)),
            pl.BlockSpec((F, 6), lambda i: (0, 0)),
            pl.BlockSpec((BI, N), lambda i: (i, 0)),
        ],
        out_specs=pl.BlockSpec((B, BI, N), lambda i: (0, i, 0)),
        out_shape=jax.ShapeDtypeStruct((B, N, N), jnp.float32),
        scratch_shapes=[
            pltpu.VMEM((B, N, 1), jnp.int32),
            pltpu.VMEM((B, 1, N), jnp.int32),
        ],
    )(node_features, port_feature_mask, dir32)
    return out
